# baseline (speedup 1.0000x reference)
import jax
import jax.numpy as jnp
from jax.experimental import pallas as pl
from jax.experimental.pallas import tpu as pltpu

_BM = 16
_DEPTH = 16


def _fused_kernel(feature_ref, w_ref, b_ref, adj_hbm, out_ref,
                  buf, sems, sup_ref, supb_ref):
    n = out_ref.shape[0]
    nchunk = n // _BM

    def copy(c, slot):
        return pltpu.make_async_copy(
            adj_hbm.at[pl.ds(c * _BM, _BM), :], buf.at[slot], sems.at[slot]
        )

    for c in range(_DEPTH):
        copy(c, c).start()

    acc = jnp.dot(feature_ref[...], w_ref[...], preferred_element_type=jnp.float32)
    sup_f32 = jnp.maximum(acc, 0.0)
    sup_ref[...] = sup_f32
    supb_ref[...] = sup_f32.astype(jnp.bfloat16)

    sup = supb_ref[...]
    bias = b_ref[...]

    def step(c, carry):
        slot = jax.lax.rem(c, _DEPTH)
        copy(c, slot).wait()
        acc = jnp.dot(
            buf[slot].astype(jnp.bfloat16), sup,
            preferred_element_type=jnp.float32,
        )

        @pl.when(c + _DEPTH < nchunk)
        def _():
            copy(c + _DEPTH, slot).start()

        out_ref[pl.ds(c * _BM, _BM), :] = (
            acc + sup_ref[pl.ds(c * _BM, _BM), :] + bias
        )
        return carry

    jax.lax.fori_loop(0, nchunk, step, 0)


def kernel(feature, adj, W, b):
    n, _ = feature.shape
    d = W.shape[1]

    out = pl.pallas_call(
        _fused_kernel,
        in_specs=[
            pl.BlockSpec(memory_space=pltpu.VMEM),
            pl.BlockSpec(memory_space=pltpu.VMEM),
            pl.BlockSpec(memory_space=pltpu.VMEM),
            pl.BlockSpec(memory_space=pltpu.HBM),
        ],
        out_specs=pl.BlockSpec(memory_space=pltpu.VMEM),
        out_shape=jax.ShapeDtypeStruct((n, d), jnp.float32),
        scratch_shapes=[
            pltpu.VMEM((_DEPTH, _BM, n), jnp.float32),
            pltpu.SemaphoreType.DMA((_DEPTH,)),
            pltpu.VMEM((n, d), jnp.float32),
            pltpu.VMEM((n, d), jnp.bfloat16),
        ],
    )(feature, W, b.reshape(1, d), adj)
    return out


# speedup vs baseline: 2.3617x; 2.3617x over previous
import jax
import jax.numpy as jnp
from jax.experimental import pallas as pl
from jax.experimental.pallas import tpu as pltpu

_BM = 80
_DEPTH = 8


def _fused_kernel(feature_ref, w_ref, b_ref, adj_hbm, out_ref,
                  buf, sems, sup_ref, supb_ref):
    n = out_ref.shape[0]
    nchunk = n // _BM

    def copy(c, slot):
        return pltpu.make_async_copy(
            adj_hbm.at[pl.ds(c * _BM, _BM), :], buf.at[slot], sems.at[slot]
        )

    for c in range(_DEPTH):
        copy(c, c).start()

    acc = jnp.dot(feature_ref[...], w_ref[...], preferred_element_type=jnp.float32)
    sup_f32 = jnp.maximum(acc, 0.0)
    sup_ref[...] = sup_f32
    supb_ref[...] = sup_f32.astype(jnp.bfloat16)

    sup = supb_ref[...]
    bias = b_ref[...]

    def step(c, carry):
        slot = jax.lax.rem(c, _DEPTH)
        copy(c, slot).wait()
        acc = jnp.dot(
            buf[slot].astype(jnp.bfloat16), sup,
            preferred_element_type=jnp.float32,
        )

        @pl.when(c + _DEPTH < nchunk)
        def _():
            copy(c + _DEPTH, slot).start()

        out_ref[pl.ds(c * _BM, _BM), :] = (
            acc + sup_ref[pl.ds(c * _BM, _BM), :] + bias
        )
        return carry

    jax.lax.fori_loop(0, nchunk, step, 0)


def kernel(feature, adj, W, b):
    n, _ = feature.shape
    d = W.shape[1]

    out = pl.pallas_call(
        _fused_kernel,
        in_specs=[
            pl.BlockSpec(memory_space=pltpu.VMEM),
            pl.BlockSpec(memory_space=pltpu.VMEM),
            pl.BlockSpec(memory_space=pltpu.VMEM),
            pl.BlockSpec(memory_space=pltpu.HBM),
        ],
        out_specs=pl.BlockSpec(memory_space=pltpu.VMEM),
        out_shape=jax.ShapeDtypeStruct((n, d), jnp.float32),
        scratch_shapes=[
            pltpu.VMEM((_DEPTH, _BM, n), jnp.float32),
            pltpu.SemaphoreType.DMA((_DEPTH,)),
            pltpu.VMEM((n, d), jnp.float32),
            pltpu.VMEM((n, d), jnp.bfloat16),
        ],
    )(feature, W, b.reshape(1, d), adj)
    return out
